# baseline (speedup 1.0000x reference)
import functools

import jax
import jax.numpy as jnp
from jax import lax
from jax.experimental import pallas as pl
from jax.experimental.pallas import tpu as pltpu
from jax.experimental.pallas import tpu_sc as plsc

_VOCAB = 1000000
_EMBED = 64
_B = 4096
_L = 200
_N = _B * _L
_LP = _L // 2

_GATHER_WINDOW = 128


def _sc_gather(token_table, flat_ids):
    mesh = plsc.VectorSubcoreMesh(core_axis_name="c", subcore_axis_name="s")

    @functools.partial(
        pl.kernel,
        out_type=jax.ShapeDtypeStruct((_N, _EMBED), jnp.float32),
        mesh=mesh,
        compiler_params=pltpu.CompilerParams(use_tc_tiling_on_sc=False),
    )
    def gather_kernel(table_hbm, idx_hbm, out_hbm):
        def body(i_vmem, o_vmem):
            pltpu.sync_copy(table_hbm.at[i_vmem.at[0]], o_vmem)

        pltpu.emit_pipeline(
            body,
            grid=(_N // _GATHER_WINDOW,),
            in_specs=[
                pl.BlockSpec((1, _GATHER_WINDOW), index_map=lambda i: (0, i))
            ],
            out_specs=[
                pl.BlockSpec((_GATHER_WINDOW, _EMBED), index_map=lambda i: (i, 0))
            ],
            core_axis_name=("c", "s"),
            dimension_semantics=(pltpu.PARALLEL,),
        )(idx_hbm, out_hbm)

    return gather_kernel(token_table, flat_ids.reshape(1, _N))


_RB = 1600


def _ln_rm_body(tok_ref, pos_ref, gamma_ref, beta_ref, out_ref):
    y = tok_ref[...] + pos_ref[...]
    y3 = y.reshape(_RB, 2, _EMBED)
    m = jnp.mean(y3, axis=2, keepdims=True)
    q = jnp.mean(y3 * y3, axis=2, keepdims=True)
    r = lax.rsqrt(q - m * m + 1e-5)
    z = ((y3 - m) * r).reshape(_RB, 128)
    out_ref[...] = z * gamma_ref[...] + beta_ref[...]


def _tc_layernorm_rm(tok2, pos_tiled, g128, b128):
    return pl.pallas_call(
        _ln_rm_body,
        grid=(_N // 2 // _RB,),
        in_specs=[
            pl.BlockSpec((_RB, 128), lambda i: (i, 0)),
            pl.BlockSpec((_RB, 128), lambda i: (0, 0)),
            pl.BlockSpec((1, 128), lambda i: (0, 0)),
            pl.BlockSpec((1, 128), lambda i: (0, 0)),
        ],
        out_specs=pl.BlockSpec((_RB, 128), lambda i: (i, 0)),
        out_shape=jax.ShapeDtypeStruct((_N // 2, 128), jnp.float32),
    )(tok2, pos_tiled, g128, b128)


def kernel(input_ids, token_table, pos_table, gamma, beta):
    flat_ids = input_ids.reshape(-1).astype(jnp.int32)
    tok = _sc_gather(token_table, flat_ids)
    tok2 = tok.reshape(_N // 2, 128)
    pos_pairs = pos_table[:_L].reshape(_LP, 128)
    pos_tiled = jnp.tile(pos_pairs, (_RB // _LP, 1))
    g128 = jnp.tile(gamma, 2).reshape(1, 128)
    b128 = jnp.tile(beta, 2).reshape(1, 128)
    out2 = _tc_layernorm_rm(tok2, pos_tiled, g128, b128)
    outF = jnp.transpose(out2.reshape(_B, _LP * 128), (1, 0))
    return jnp.transpose(outF.reshape(_L, _EMBED, _B), (2, 0, 1))
